# baseline (speedup 1.0000x reference)
import jax
import jax.numpy as jnp
from jax import lax
from jax.experimental import pallas as pl
from jax.experimental.pallas import tpu as pltpu
from jax.experimental.pallas import tpu_sc as plsc

_MASKED_TOKEN = 0
_NUM_CORES = 2
_NUM_SUBCORES = 16
_NUM_WORKERS = _NUM_CORES * _NUM_SUBCORES
_CB = 2
_LANES = 16


def _gather_body(idx_hbm, table_hbm, out_hbm, idx_v, stage_v, sem):
    nb, t = idx_hbm.shape
    d = table_hbm.shape[1]
    rows_per_w = nb // _NUM_WORKERS
    n_chunks = rows_per_w // _CB
    wid = lax.axis_index("s") * _NUM_CORES + lax.axis_index("c")
    w_base = wid * rows_per_w

    lane = lax.iota(jnp.int32, _LANES)
    zeros16 = jnp.zeros((_LANES,), jnp.float32)
    n_groups = (t + _LANES - 1) // _LANES
    last_off = t - _LANES

    def chunk_step(i, carry):
        b0 = w_base + i * _CB
        for b in range(_CB):
            pltpu.sync_copy(idx_hbm.at[b0 + b], idx_v.at[b])
        copies = []
        for b in range(_CB):
            copies.append(pltpu.async_copy(
                table_hbm.at[idx_v.at[b, pl.ds(0, 128)]],
                stage_v.at[b, pl.ds(0, 128)], sem))
            copies.append(pltpu.async_copy(
                table_hbm.at[idx_v.at[b, pl.ds(128, t - 128)]],
                stage_v.at[b, pl.ds(128, t - 128)], sem))
        for cp in copies:
            cp.wait()

        for b in range(_CB):
            def group_step(g, carry2, b=b):
                off = jnp.minimum(g * _LANES, last_off)
                vec = idx_v[b, pl.ds(off, _LANES)]
                m = vec == _MASKED_TOKEN

                @pl.when(jnp.any(m))
                def _():
                    rows = off + lane
                    for j in range(d):
                        plsc.store_scatter(
                            stage_v.at[b],
                            [rows, jnp.full((_LANES,), j, jnp.int32)],
                            zeros16,
                            mask=m,
                        )

                return carry2

            lax.fori_loop(0, n_groups, group_step, 0)

        pltpu.sync_copy(stage_v, out_hbm.at[pl.ds(b0, _CB)])
        return carry

    lax.fori_loop(0, n_chunks, chunk_step, 0)


def kernel(indices, embed):
    nb, t = indices.shape
    d = embed.shape[1]
    mesh = plsc.VectorSubcoreMesh(
        core_axis_name="c",
        subcore_axis_name="s",
        num_cores=_NUM_CORES,
        num_subcores=_NUM_SUBCORES,
    )
    run = pl.kernel(
        _gather_body,
        out_type=jax.ShapeDtypeStruct((nb, t, d), jnp.float32),
        mesh=mesh,
        scratch_types=[
            pltpu.VMEM((_CB, t), jnp.int32),
            pltpu.VMEM((_CB, t, d), jnp.float32),
            pltpu.SemaphoreType.DMA,
        ],
        compiler_params=pltpu.CompilerParams(
            needs_layout_passes=False, use_tc_tiling_on_sc=False
        ),
    )
    return run(indices.astype(jnp.int32), embed)
